# Initial kernel scaffold; baseline (speedup 1.0000x reference)
import jax
import jax.numpy as jnp
from jax.experimental import pallas as pl


def kernel(target_x, target_y, mu):
    raise NotImplementedError("write your pallas kernel here")


# speedup vs baseline: 14.6069x; 14.6069x over previous
import functools

import jax
import jax.numpy as jnp
import numpy as np
from jax.experimental import pallas as pl

N = 8192
K = 32
DECAY = 0.1
ROWS = 256
LO_BITS = int(np.float32(1e-12).view(np.int32))
HI_BITS = int(np.float32(4.0).view(np.int32))
BISECT_ITERS = 30


def _moran_kernel(xr, yr, sar, tyr, mur,
                  xc, yc, sac, tyc, muc,
                  num1_o, num2_o, sumw_o, den1_o, den2_o):
    pid = pl.program_id(0)

    mean_y = jnp.sum(tyc[...]) / N
    mean_m = jnp.sum(muc[...]) / N
    devy_c = tyc[...] - mean_y
    devm_c = muc[...] - mean_m
    devy_r = tyr[...] - mean_y
    devm_r = mur[...] - mean_m

    fxr = xr[...].astype(jnp.float32)
    fyr = yr[...].astype(jnp.float32)
    fxc = xc[...].astype(jnp.float32)
    fyc = yc[...].astype(jnp.float32)
    ab = fxr * fxc + fyr * fyc
    d2 = (sar[...] + sac[...]) - 2.0 * ab
    d2c = jnp.maximum(d2, 1e-12)

    r = d2c.shape[0]
    lo0 = jnp.full((r, 1), LO_BITS, dtype=jnp.int32)
    hi0 = jnp.full((r, 1), HI_BITS, dtype=jnp.int32)

    def body(_, carry):
        lo, hi = carry
        mid = lo + (hi - lo) // 2
        t = jax.lax.bitcast_convert_type(mid, jnp.float32)
        cnt = jnp.sum(jnp.where(d2c <= t, 1.0, 0.0), axis=1, keepdims=True)
        pred = cnt >= K
        hi = jnp.where(pred, mid, hi)
        lo = jnp.where(pred, lo, mid + 1)
        return lo, hi

    lo, _ = jax.lax.fori_loop(0, BISECT_ITERS, body, (lo0, hi0))
    t = jax.lax.bitcast_convert_type(lo, jnp.float32)

    lt = d2c < t
    eq = d2c == t
    cnt_lt = jnp.sum(jnp.where(lt, 1.0, 0.0), axis=1, keepdims=True)
    m = K - cnt_lt

    cols = jax.lax.broadcasted_iota(jnp.int32, (1, N), 1)
    jlo0 = jnp.zeros((r, 1), dtype=jnp.int32)
    jhi0 = jnp.full((r, 1), N - 1, dtype=jnp.int32)

    def jbody(_, carry):
        jlo, jhi = carry
        jmid = jlo + (jhi - jlo) // 2
        c = jnp.sum(jnp.where(eq & (cols <= jmid), 1.0, 0.0),
                    axis=1, keepdims=True)
        p = c >= m
        return jnp.where(p, jlo, jmid + 1), jnp.where(p, jmid, jhi)

    jsel, _ = jax.lax.fori_loop(0, 13, jbody, (jlo0, jhi0))
    eq_sel = eq & (cols <= jsel)

    w = jnp.exp(-DECAY * jnp.sqrt(d2c))
    w_lt = jnp.where(lt, w, 0.0)
    w_t = jnp.exp(-DECAY * jnp.sqrt(t))

    sw_rows = jnp.sum(w_lt, axis=1, keepdims=True) + m * w_t

    sum_eq_y = jnp.sum(jnp.where(eq_sel, devy_c, 0.0), axis=1, keepdims=True)
    sdy = jnp.sum(w_lt * devy_c, axis=1, keepdims=True) + w_t * sum_eq_y
    sum_eq_m = jnp.sum(jnp.where(eq_sel, devm_c, 0.0), axis=1, keepdims=True)
    sdm = jnp.sum(w_lt * devm_c, axis=1, keepdims=True) + w_t * sum_eq_m

    num1_blk = jnp.sum(devy_r * sdy).reshape(1, 1)
    num2_blk = jnp.sum(devm_r * sdm).reshape(1, 1)
    sumw_blk = jnp.sum(sw_rows).reshape(1, 1)

    den1_o[...] = jnp.sum(devy_c * devy_c).reshape(1, 1)
    den2_o[...] = jnp.sum(devm_c * devm_c).reshape(1, 1)

    @pl.when(pid == 0)
    def _():
        num1_o[...] = num1_blk
        num2_o[...] = num2_blk
        sumw_o[...] = sumw_blk

    @pl.when(pid != 0)
    def _():
        num1_o[...] += num1_blk
        num2_o[...] += num2_blk
        sumw_o[...] += sumw_blk


@jax.jit
def kernel(target_x, target_y, mu):
    tx = jnp.squeeze(target_x, axis=0)
    ty = jnp.squeeze(jnp.squeeze(target_y, axis=0), axis=-1)
    m = jnp.squeeze(jnp.squeeze(mu, axis=0), axis=-1)
    x = tx[:, 0]
    y = tx[:, 1]
    sa = x * x + y * y
    bx = x.astype(jnp.bfloat16)
    by = y.astype(jnp.bfloat16)

    col = lambda v: v.reshape(1, N)
    row = lambda v: v.reshape(N, 1)

    grid = N // ROWS
    row_spec = pl.BlockSpec((ROWS, 1), lambda i: (i, 0))
    col_spec = pl.BlockSpec((1, N), lambda i: (0, 0))
    out_spec = pl.BlockSpec((1, 1), lambda i: (0, 0))
    scalar = jax.ShapeDtypeStruct((1, 1), jnp.float32)

    num1, num2, sumw, den1, den2 = pl.pallas_call(
        _moran_kernel,
        grid=(grid,),
        in_specs=[row_spec] * 5 + [col_spec] * 5,
        out_specs=[out_spec] * 5,
        out_shape=[scalar] * 5,
    )(row(bx), row(by), row(sa), row(ty), row(m),
      col(bx), col(by), col(sa), col(ty), col(m))

    nf = jnp.float32(N)
    moran_y = nf / sumw[0, 0] * num1[0, 0] / den1[0, 0]
    moran_mu = nf / sumw[0, 0] * num2[0, 0] / den2[0, 0]
    return (moran_y, moran_mu)


# speedup vs baseline: 15.8239x; 1.0833x over previous
import functools

import jax
import jax.numpy as jnp
import numpy as np
from jax.experimental import pallas as pl

N = 8192
K = 32
DECAY = 0.1
ROWS = 512
LO_BITS = int(np.float32(1e-12).view(np.int32))
HI_BITS = int(np.float32(4.0).view(np.int32))
BISECT_ITERS = 29

_E4 = 3.7627182874387473e-06
_E3 = -0.00016595564079941984
_E2 = 0.004999484330249198
_E1 = -0.09999986860734106
_E0 = 0.9999999946332208


def _wexp(s):
    return (((_E4 * s + _E3) * s + _E2) * s + _E1) * s + _E0


def _moran_kernel(xr, yr, sar, tyr, mur,
                  xc, yc, sac, tyc, muc,
                  num1_o, num2_o, sumw_o, den1_o, den2_o):
    pid = pl.program_id(0)

    mean_y = jnp.sum(tyc[...]) / N
    mean_m = jnp.sum(muc[...]) / N
    devy_c = tyc[...] - mean_y
    devm_c = muc[...] - mean_m
    devy_r = tyr[...] - mean_y
    devm_r = mur[...] - mean_m

    fxr = xr[...].astype(jnp.float32)
    fyr = yr[...].astype(jnp.float32)
    fxc = xc[...].astype(jnp.float32)
    fyc = yc[...].astype(jnp.float32)
    ab = fxr * fxc + fyr * fyc
    d2 = (sar[...] + sac[...]) - 2.0 * ab
    d2c = jnp.maximum(d2, 1e-12)

    r = d2c.shape[0]
    lo0 = jnp.full((r, 1), LO_BITS, dtype=jnp.int32)
    hi0 = jnp.full((r, 1), HI_BITS, dtype=jnp.int32)

    def body(_, carry):
        lo, hi = carry
        mid = lo + (hi - lo) // 2
        t = jax.lax.bitcast_convert_type(mid, jnp.float32)
        cnt = jnp.sum(jnp.where(d2c <= t, 1.0, 0.0), axis=1, keepdims=True)
        pred = cnt >= K
        hi = jnp.where(pred, mid, hi)
        lo = jnp.where(pred, lo, mid + 1)
        return lo, hi

    lo, _ = jax.lax.fori_loop(0, BISECT_ITERS, body, (lo0, hi0))
    t = jax.lax.bitcast_convert_type(lo, jnp.float32)

    lt = d2c < t
    eq = d2c == t
    cnt_lt = jnp.sum(jnp.where(lt, 1.0, 0.0), axis=1, keepdims=True)
    m = K - cnt_lt

    cols = jax.lax.broadcasted_iota(jnp.int32, (1, N), 1)
    eqcol = jnp.where(eq, cols, N)
    jlo0 = jnp.zeros((r, 1), dtype=jnp.int32)
    jhi0 = jnp.full((r, 1), N - 1, dtype=jnp.int32)

    def jbody(_, carry):
        jlo, jhi = carry
        jmid = jlo + (jhi - jlo) // 2
        c = jnp.sum(jnp.where(eqcol <= jmid, 1.0, 0.0),
                    axis=1, keepdims=True)
        p = c >= m
        return jnp.where(p, jlo, jmid + 1), jnp.where(p, jmid, jhi)

    jsel, _ = jax.lax.fori_loop(0, 13, jbody, (jlo0, jhi0))
    eq_sel = eqcol <= jsel

    w = _wexp(jnp.sqrt(d2c))
    w_lt = jnp.where(lt, w, 0.0)
    w_t = _wexp(jnp.sqrt(t))

    sw_rows = jnp.sum(w_lt, axis=1, keepdims=True) + m * w_t

    sum_eq_y = jnp.sum(jnp.where(eq_sel, devy_c, 0.0), axis=1, keepdims=True)
    sdy = jnp.sum(w_lt * devy_c, axis=1, keepdims=True) + w_t * sum_eq_y
    sum_eq_m = jnp.sum(jnp.where(eq_sel, devm_c, 0.0), axis=1, keepdims=True)
    sdm = jnp.sum(w_lt * devm_c, axis=1, keepdims=True) + w_t * sum_eq_m

    num1_blk = jnp.sum(devy_r * sdy).reshape(1, 1)
    num2_blk = jnp.sum(devm_r * sdm).reshape(1, 1)
    sumw_blk = jnp.sum(sw_rows).reshape(1, 1)

    den1_o[...] = jnp.sum(devy_c * devy_c).reshape(1, 1)
    den2_o[...] = jnp.sum(devm_c * devm_c).reshape(1, 1)

    @pl.when(pid == 0)
    def _():
        num1_o[...] = num1_blk
        num2_o[...] = num2_blk
        sumw_o[...] = sumw_blk

    @pl.when(pid != 0)
    def _():
        num1_o[...] += num1_blk
        num2_o[...] += num2_blk
        sumw_o[...] += sumw_blk


@jax.jit
def kernel(target_x, target_y, mu):
    tx = jnp.squeeze(target_x, axis=0)
    ty = jnp.squeeze(jnp.squeeze(target_y, axis=0), axis=-1)
    m = jnp.squeeze(jnp.squeeze(mu, axis=0), axis=-1)
    x = tx[:, 0]
    y = tx[:, 1]
    sa = x * x + y * y
    bx = x.astype(jnp.bfloat16)
    by = y.astype(jnp.bfloat16)

    col = lambda v: v.reshape(1, N)
    row = lambda v: v.reshape(N, 1)

    grid = N // ROWS
    row_spec = pl.BlockSpec((ROWS, 1), lambda i: (i, 0))
    col_spec = pl.BlockSpec((1, N), lambda i: (0, 0))
    out_spec = pl.BlockSpec((1, 1), lambda i: (0, 0))
    scalar = jax.ShapeDtypeStruct((1, 1), jnp.float32)

    num1, num2, sumw, den1, den2 = pl.pallas_call(
        _moran_kernel,
        grid=(grid,),
        in_specs=[row_spec] * 5 + [col_spec] * 5,
        out_specs=[out_spec] * 5,
        out_shape=[scalar] * 5,
    )(row(bx), row(by), row(sa), row(ty), row(m),
      col(bx), col(by), col(sa), col(ty), col(m))

    nf = jnp.float32(N)
    moran_y = nf / sumw[0, 0] * num1[0, 0] / den1[0, 0]
    moran_mu = nf / sumw[0, 0] * num2[0, 0] / den2[0, 0]
    return (moran_y, moran_mu)


# speedup vs baseline: 17.5933x; 1.1118x over previous
import functools

import jax
import jax.numpy as jnp
import numpy as np
from jax.experimental import pallas as pl

N = 8192
K = 32
DECAY = 0.1
ROWS = 512
LO_BITS = int(np.float32(1e-12).view(np.int32))
HI_BITS = int(np.float32(4.0).view(np.int32))
BISECT_ITERS = 29

_E4 = 3.7627182874387473e-06
_E3 = -0.00016595564079941984
_E2 = 0.004999484330249198
_E1 = -0.09999986860734106
_E0 = 0.9999999946332208


def _wexp(s):
    return (((_E4 * s + _E3) * s + _E2) * s + _E1) * s + _E0


def _moran_kernel(xr, yr, sar, tyr, mur,
                  xc, yc, sac, tyc, muc,
                  num1_o, num2_o, sumw_o, den1_o, den2_o):
    pid = pl.program_id(0)

    mean_y = jnp.sum(tyc[...]) / N
    mean_m = jnp.sum(muc[...]) / N
    devy_c = tyc[...] - mean_y
    devm_c = muc[...] - mean_m
    devy_r = tyr[...] - mean_y
    devm_r = mur[...] - mean_m

    fxr = xr[...].astype(jnp.float32)
    fyr = yr[...].astype(jnp.float32)
    fxc = xc[...].astype(jnp.float32)
    fyc = yc[...].astype(jnp.float32)
    ab = fxr * fxc + fyr * fyc
    d2 = (sar[...] + sac[...]) - 2.0 * ab
    d2c = jnp.maximum(d2, 1e-12)

    r = d2c.shape[0]
    bits = jax.lax.bitcast_convert_type(d2c, jnp.int32)
    k_hi = (bits >> 16).astype(jnp.int16)

    one_b = jnp.bfloat16(1)
    zero_b = jnp.bfloat16(0)

    lo0 = jnp.full((r, 1), LO_BITS >> 16, dtype=jnp.int32)
    hi0 = jnp.full((r, 1), HI_BITS >> 16, dtype=jnp.int32)

    def body1(_, carry):
        lo, hi = carry
        mid = lo + (hi - lo) // 2
        mid16 = mid.astype(jnp.int16)
        cnt = jnp.sum(jnp.where(k_hi <= mid16, one_b, zero_b),
                      axis=1, keepdims=True, dtype=jnp.bfloat16)
        pred = cnt.astype(jnp.float32) >= K
        return jnp.where(pred, lo, mid + 1), jnp.where(pred, mid, hi)

    v, _ = jax.lax.fori_loop(0, 13, body1, (lo0, hi0))
    v16 = v.astype(jnp.int16)

    cnt_below = jnp.sum(jnp.where(k_hi < v16, one_b, zero_b),
                        axis=1, keepdims=True,
                        dtype=jnp.bfloat16).astype(jnp.float32)
    target2 = K - cnt_below

    low16 = ((bits & 0xFFFF) - 32768).astype(jnp.int16)
    sel_low = jnp.where(k_hi == v16, low16, jnp.int16(32767))

    lo20 = jnp.full((r, 1), -32768, dtype=jnp.int32)
    hi20 = jnp.full((r, 1), 32767, dtype=jnp.int32)

    def body2(_, carry):
        lo, hi = carry
        mid = lo + (hi - lo) // 2
        mid16 = mid.astype(jnp.int16)
        cnt = jnp.sum(jnp.where(sel_low <= mid16, one_b, zero_b),
                      axis=1, keepdims=True, dtype=jnp.bfloat16)
        pred = cnt.astype(jnp.float32) >= target2
        return jnp.where(pred, lo, mid + 1), jnp.where(pred, mid, hi)

    lo2, _ = jax.lax.fori_loop(0, 16, body2, (lo20, hi20))
    t_bits = (v << 16) | ((lo2 + 32768) & 0xFFFF)
    t = jax.lax.bitcast_convert_type(t_bits, jnp.float32)

    lt = d2c < t
    cnt_lt = jnp.sum(jnp.where(lt, 1.0, 0.0), axis=1, keepdims=True)
    m = K - cnt_lt

    lo2_16 = lo2.astype(jnp.int16)
    eq16 = (sel_low == lo2_16) & (k_hi == v16)
    cols16 = jax.lax.broadcasted_iota(jnp.int16, (1, N), 1)
    eqcol = jnp.where(eq16, cols16, jnp.int16(N))
    jlo0 = jnp.zeros((r, 1), dtype=jnp.int32)
    jhi0 = jnp.full((r, 1), N - 1, dtype=jnp.int32)

    def jbody(_, carry):
        jlo, jhi = carry
        jmid = jlo + (jhi - jlo) // 2
        jmid16 = jmid.astype(jnp.int16)
        c = jnp.sum(jnp.where(eqcol <= jmid16, one_b, zero_b),
                    axis=1, keepdims=True, dtype=jnp.bfloat16)
        p = c.astype(jnp.float32) >= m
        return jnp.where(p, jlo, jmid + 1), jnp.where(p, jmid, jhi)

    jsel, _ = jax.lax.fori_loop(0, 13, jbody, (jlo0, jhi0))
    colsf = jax.lax.broadcasted_iota(jnp.int32, (1, N), 1).astype(jnp.float32)
    eq_sel = (d2c == t) & (colsf <= jsel.astype(jnp.float32))

    w = _wexp(jnp.sqrt(d2c))
    w_lt = jnp.where(lt, w, 0.0)
    w_t = _wexp(jnp.sqrt(t))

    sw_rows = jnp.sum(w_lt, axis=1, keepdims=True) + m * w_t

    sum_eq_y = jnp.sum(jnp.where(eq_sel, devy_c, 0.0), axis=1, keepdims=True)
    sdy = jnp.sum(w_lt * devy_c, axis=1, keepdims=True) + w_t * sum_eq_y
    sum_eq_m = jnp.sum(jnp.where(eq_sel, devm_c, 0.0), axis=1, keepdims=True)
    sdm = jnp.sum(w_lt * devm_c, axis=1, keepdims=True) + w_t * sum_eq_m

    num1_blk = jnp.sum(devy_r * sdy).reshape(1, 1)
    num2_blk = jnp.sum(devm_r * sdm).reshape(1, 1)
    sumw_blk = jnp.sum(sw_rows).reshape(1, 1)

    den1_o[...] = jnp.sum(devy_c * devy_c).reshape(1, 1)
    den2_o[...] = jnp.sum(devm_c * devm_c).reshape(1, 1)

    @pl.when(pid == 0)
    def _():
        num1_o[...] = num1_blk
        num2_o[...] = num2_blk
        sumw_o[...] = sumw_blk

    @pl.when(pid != 0)
    def _():
        num1_o[...] += num1_blk
        num2_o[...] += num2_blk
        sumw_o[...] += sumw_blk


@jax.jit
def kernel(target_x, target_y, mu):
    tx = jnp.squeeze(target_x, axis=0)
    ty = jnp.squeeze(jnp.squeeze(target_y, axis=0), axis=-1)
    m = jnp.squeeze(jnp.squeeze(mu, axis=0), axis=-1)
    x = tx[:, 0]
    y = tx[:, 1]
    sa = x * x + y * y
    bx = x.astype(jnp.bfloat16)
    by = y.astype(jnp.bfloat16)

    col = lambda v: v.reshape(1, N)
    row = lambda v: v.reshape(N, 1)

    grid = N // ROWS
    row_spec = pl.BlockSpec((ROWS, 1), lambda i: (i, 0))
    col_spec = pl.BlockSpec((1, N), lambda i: (0, 0))
    out_spec = pl.BlockSpec((1, 1), lambda i: (0, 0))
    scalar = jax.ShapeDtypeStruct((1, 1), jnp.float32)

    num1, num2, sumw, den1, den2 = pl.pallas_call(
        _moran_kernel,
        grid=(grid,),
        in_specs=[row_spec] * 5 + [col_spec] * 5,
        out_specs=[out_spec] * 5,
        out_shape=[scalar] * 5,
    )(row(bx), row(by), row(sa), row(ty), row(m),
      col(bx), col(by), col(sa), col(ty), col(m))

    nf = jnp.float32(N)
    moran_y = nf / sumw[0, 0] * num1[0, 0] / den1[0, 0]
    moran_mu = nf / sumw[0, 0] * num2[0, 0] / den2[0, 0]
    return (moran_y, moran_mu)


# speedup vs baseline: 17.9884x; 1.0225x over previous
import functools

import jax
import jax.numpy as jnp
import numpy as np
from jax.experimental import pallas as pl

N = 8192
K = 32
DECAY = 0.1
ROWS = 512
LO_BITS = int(np.float32(1e-12).view(np.int32))
HI_BITS = int(np.float32(4.0).view(np.int32))
BISECT_ITERS = 29

_E4 = 3.7627182874387473e-06
_E3 = -0.00016595564079941984
_E2 = 0.004999484330249198
_E1 = -0.09999986860734106
_E0 = 0.9999999946332208


def _wexp(s):
    return (((_E4 * s + _E3) * s + _E2) * s + _E1) * s + _E0


def _moran_kernel(xr, yr, sar, tyr, mur,
                  xc, yc, sac, tyc, muc,
                  num1_o, num2_o, sumw_o, den1_o, den2_o):
    pid = pl.program_id(0)

    mean_y = jnp.sum(tyc[...]) / N
    mean_m = jnp.sum(muc[...]) / N
    devy_c = tyc[...] - mean_y
    devm_c = muc[...] - mean_m
    devy_r = tyr[...] - mean_y
    devm_r = mur[...] - mean_m

    fxr = xr[...].astype(jnp.float32)
    fyr = yr[...].astype(jnp.float32)
    fxc = xc[...].astype(jnp.float32)
    fyc = yc[...].astype(jnp.float32)
    ab = fxr * fxc + fyr * fyc
    d2 = (sar[...] + sac[...]) - 2.0 * ab
    d2c = jnp.maximum(d2, 1e-12)

    r = d2c.shape[0]
    bits = jax.lax.bitcast_convert_type(d2c, jnp.int32)
    k_hi = (bits >> 16).astype(jnp.int16)

    one_b = jnp.bfloat16(1)
    zero_b = jnp.bfloat16(0)

    lo0 = jnp.full((r, 1), LO_BITS >> 16, dtype=jnp.int32)
    hi0 = jnp.full((r, 1), HI_BITS >> 16, dtype=jnp.int32)

    def body1(_, carry):
        lo, hi = carry
        mid = lo + (hi - lo) // 2
        mid16 = mid.astype(jnp.int16)
        cnt = jnp.sum(jnp.where(k_hi <= mid16, one_b, zero_b),
                      axis=1, keepdims=True, dtype=jnp.bfloat16)
        pred = cnt.astype(jnp.float32) >= K
        return jnp.where(pred, lo, mid + 1), jnp.where(pred, mid, hi)

    v, _ = jax.lax.fori_loop(0, 13, body1, (lo0, hi0))
    v16 = v.astype(jnp.int16)

    cnt_below = jnp.sum(jnp.where(k_hi < v16, one_b, zero_b),
                        axis=1, keepdims=True,
                        dtype=jnp.bfloat16).astype(jnp.float32)
    target2 = K - cnt_below

    low16 = (bits ^ 0x8000).astype(jnp.int16)
    sel_low = jnp.where(k_hi == v16, low16, jnp.int16(32767))

    lo20 = jnp.full((r, 1), -32768, dtype=jnp.int32)
    hi20 = jnp.full((r, 1), 32767, dtype=jnp.int32)

    def body2(_, carry):
        lo, hi = carry
        mid = lo + (hi - lo) // 2
        mid16 = mid.astype(jnp.int16)
        cnt = jnp.sum(jnp.where(sel_low <= mid16, one_b, zero_b),
                      axis=1, keepdims=True, dtype=jnp.bfloat16)
        pred = cnt.astype(jnp.float32) >= target2
        return jnp.where(pred, lo, mid + 1), jnp.where(pred, mid, hi)

    lo2, _ = jax.lax.fori_loop(0, 16, body2, (lo20, hi20))
    t_bits = (v << 16) | ((lo2 + 32768) & 0xFFFF)
    t = jax.lax.bitcast_convert_type(t_bits, jnp.float32)

    lt = d2c < t
    lo2_16s = lo2.astype(jnp.int16)
    cnt_in = jnp.sum(jnp.where(sel_low < lo2_16s, one_b, zero_b),
                     axis=1, keepdims=True, dtype=jnp.bfloat16)
    cnt_lt = cnt_below + cnt_in.astype(jnp.float32)
    m = K - cnt_lt

    lo2_16 = lo2.astype(jnp.int16)
    eq16 = (sel_low == lo2_16) & (k_hi == v16)
    cols16 = jax.lax.broadcasted_iota(jnp.int16, (1, N), 1)
    eqcol = jnp.where(eq16, cols16, jnp.int16(N))
    jlo0 = jnp.zeros((r, 1), dtype=jnp.int32)
    jhi0 = jnp.full((r, 1), N - 1, dtype=jnp.int32)

    def jbody(_, carry):
        jlo, jhi = carry
        jmid = jlo + (jhi - jlo) // 2
        jmid16 = jmid.astype(jnp.int16)
        c = jnp.sum(jnp.where(eqcol <= jmid16, one_b, zero_b),
                    axis=1, keepdims=True, dtype=jnp.bfloat16)
        p = c.astype(jnp.float32) >= m
        return jnp.where(p, jlo, jmid + 1), jnp.where(p, jmid, jhi)

    jsel, _ = jax.lax.fori_loop(0, 13, jbody, (jlo0, jhi0))
    colsf = jax.lax.broadcasted_iota(jnp.int32, (1, N), 1).astype(jnp.float32)
    eq_sel = (d2c == t) & (colsf <= jsel.astype(jnp.float32))

    w = _wexp(jnp.sqrt(d2c))
    w_lt = jnp.where(lt, w, 0.0)
    w_t = _wexp(jnp.sqrt(t))

    sw_rows = jnp.sum(w_lt, axis=1, keepdims=True) + m * w_t

    sum_eq_y = jnp.sum(jnp.where(eq_sel, devy_c, 0.0), axis=1, keepdims=True)
    sdy = jnp.sum(w_lt * devy_c, axis=1, keepdims=True) + w_t * sum_eq_y
    sum_eq_m = jnp.sum(jnp.where(eq_sel, devm_c, 0.0), axis=1, keepdims=True)
    sdm = jnp.sum(w_lt * devm_c, axis=1, keepdims=True) + w_t * sum_eq_m

    num1_blk = jnp.sum(devy_r * sdy).reshape(1, 1)
    num2_blk = jnp.sum(devm_r * sdm).reshape(1, 1)
    sumw_blk = jnp.sum(sw_rows).reshape(1, 1)

    den1_o[...] = jnp.sum(devy_c * devy_c).reshape(1, 1)
    den2_o[...] = jnp.sum(devm_c * devm_c).reshape(1, 1)

    @pl.when(pid == 0)
    def _():
        num1_o[...] = num1_blk
        num2_o[...] = num2_blk
        sumw_o[...] = sumw_blk

    @pl.when(pid != 0)
    def _():
        num1_o[...] += num1_blk
        num2_o[...] += num2_blk
        sumw_o[...] += sumw_blk


@jax.jit
def kernel(target_x, target_y, mu):
    tx = jnp.squeeze(target_x, axis=0)
    ty = jnp.squeeze(jnp.squeeze(target_y, axis=0), axis=-1)
    m = jnp.squeeze(jnp.squeeze(mu, axis=0), axis=-1)
    x = tx[:, 0]
    y = tx[:, 1]
    sa = x * x + y * y
    bx = x.astype(jnp.bfloat16)
    by = y.astype(jnp.bfloat16)

    col = lambda v: v.reshape(1, N)
    row = lambda v: v.reshape(N, 1)

    grid = N // ROWS
    row_spec = pl.BlockSpec((ROWS, 1), lambda i: (i, 0))
    col_spec = pl.BlockSpec((1, N), lambda i: (0, 0))
    out_spec = pl.BlockSpec((1, 1), lambda i: (0, 0))
    scalar = jax.ShapeDtypeStruct((1, 1), jnp.float32)

    num1, num2, sumw, den1, den2 = pl.pallas_call(
        _moran_kernel,
        grid=(grid,),
        in_specs=[row_spec] * 5 + [col_spec] * 5,
        out_specs=[out_spec] * 5,
        out_shape=[scalar] * 5,
    )(row(bx), row(by), row(sa), row(ty), row(m),
      col(bx), col(by), col(sa), col(ty), col(m))

    nf = jnp.float32(N)
    moran_y = nf / sumw[0, 0] * num1[0, 0] / den1[0, 0]
    moran_mu = nf / sumw[0, 0] * num2[0, 0] / den2[0, 0]
    return (moran_y, moran_mu)


# speedup vs baseline: 19.9800x; 1.1107x over previous
import functools

import jax
import jax.numpy as jnp
import numpy as np
from jax.experimental import pallas as pl

N = 8192
K = 32
DECAY = 0.1
ROWS = 512
LO_BITS = int(np.float32(1e-12).view(np.int32))
HI_BITS = int(np.float32(4.0).view(np.int32))
BISECT_ITERS = 29

_E4 = 3.7627182874387473e-06
_E3 = -0.00016595564079941984
_E2 = 0.004999484330249198
_E1 = -0.09999986860734106
_E0 = 0.9999999946332208


def _wexp(s):
    return (((_E4 * s + _E3) * s + _E2) * s + _E1) * s + _E0


def _moran_kernel(xr, yr, sar, tyr, mur,
                  xc, yc, sac, tyc, muc,
                  num1_o, num2_o, sumw_o, den1_o, den2_o):
    pid = pl.program_id(0)

    mean_y = jnp.sum(tyc[...]) / N
    mean_m = jnp.sum(muc[...]) / N
    devy_c = tyc[...] - mean_y
    devm_c = muc[...] - mean_m
    devy_r = tyr[...] - mean_y
    devm_r = mur[...] - mean_m

    fxr = xr[...].astype(jnp.float32)
    fyr = yr[...].astype(jnp.float32)
    fxc = xc[...].astype(jnp.float32)
    fyc = yc[...].astype(jnp.float32)
    ab = fxr * fxc + fyr * fyc
    d2 = (sar[...] + sac[...]) - 2.0 * ab
    d2c = jnp.maximum(d2, 1e-12)

    r = d2c.shape[0]
    bits = jax.lax.bitcast_convert_type(d2c, jnp.int32)
    k_hi = (bits >> 16).astype(jnp.int16)

    one_b = jnp.bfloat16(1)
    zero_b = jnp.bfloat16(0)

    lo0 = jnp.full((r, 1), LO_BITS >> 16, dtype=jnp.int32)
    hi0 = jnp.full((r, 1), HI_BITS >> 16, dtype=jnp.int32)

    def body1(_, carry):
        lo, hi = carry
        mid = lo + (hi - lo) // 2
        mid16 = mid.astype(jnp.int16)
        cnt = jnp.sum(jnp.where(k_hi <= mid16, one_b, zero_b),
                      axis=1, keepdims=True, dtype=jnp.bfloat16)
        pred = cnt.astype(jnp.float32) >= K
        return jnp.where(pred, lo, mid + 1), jnp.where(pred, mid, hi)

    v, _ = jax.lax.fori_loop(0, 13, body1, (lo0, hi0))
    v16 = v.astype(jnp.int16)

    cnt_below = jnp.sum(jnp.where(k_hi < v16, one_b, zero_b),
                        axis=1, keepdims=True,
                        dtype=jnp.bfloat16).astype(jnp.float32)
    target2 = K - cnt_below

    low16 = (bits ^ 0x8000).astype(jnp.int16)
    sel_low = jnp.where(k_hi == v16, low16, jnp.int16(32767))

    lo20 = jnp.full((r, 1), -32768, dtype=jnp.int32)
    hi20 = jnp.full((r, 1), 32767, dtype=jnp.int32)

    def body2(_, carry):
        lo, hi = carry
        mid = lo + (hi - lo) // 2
        mid16 = mid.astype(jnp.int16)
        cnt = jnp.sum(jnp.where(sel_low <= mid16, one_b, zero_b),
                      axis=1, keepdims=True, dtype=jnp.bfloat16)
        pred = cnt.astype(jnp.float32) >= target2
        return jnp.where(pred, lo, mid + 1), jnp.where(pred, mid, hi)

    lo2, _ = jax.lax.fori_loop(0, 16, body2, (lo20, hi20))
    t_bits = (v << 16) | ((lo2 + 32768) & 0xFFFF)
    t = jax.lax.bitcast_convert_type(t_bits, jnp.float32)

    lt = d2c < t
    lo2_16s = lo2.astype(jnp.int16)
    cnt_in = jnp.sum(jnp.where(sel_low < lo2_16s, one_b, zero_b),
                     axis=1, keepdims=True, dtype=jnp.bfloat16)
    cnt_lt = cnt_below + cnt_in.astype(jnp.float32)
    m = K - cnt_lt

    lo2_16 = lo2.astype(jnp.int16)
    eq16 = (sel_low == lo2_16) & (k_hi == v16)
    cols16 = jax.lax.broadcasted_iota(jnp.int16, (1, N), 1)
    eqcol = jnp.where(eq16, cols16, jnp.int16(N))
    jlo0 = jnp.zeros((r, 1), dtype=jnp.int32)
    jhi0 = jnp.full((r, 1), N - 1, dtype=jnp.int32)

    def jbody(_, carry):
        jlo, jhi = carry
        jmid = jlo + (jhi - jlo) // 2
        jmid16 = jmid.astype(jnp.int16)
        c = jnp.sum(jnp.where(eqcol <= jmid16, one_b, zero_b),
                    axis=1, keepdims=True, dtype=jnp.bfloat16)
        p = c.astype(jnp.float32) >= m
        return jnp.where(p, jlo, jmid + 1), jnp.where(p, jmid, jhi)

    jsel, _ = jax.lax.fori_loop(0, 13, jbody, (jlo0, jhi0))
    colsf = jax.lax.broadcasted_iota(jnp.int32, (1, N), 1).astype(jnp.float32)
    sel = lt | ((d2c == t) & (colsf <= jsel.astype(jnp.float32)))

    w = _wexp(jnp.sqrt(d2c))
    w_sel = jnp.where(sel, w, 0.0)

    sw_rows = jnp.sum(w_sel, axis=1, keepdims=True)
    sdy = jnp.sum(w_sel * devy_c, axis=1, keepdims=True)
    sdm = jnp.sum(w_sel * devm_c, axis=1, keepdims=True)

    num1_blk = jnp.sum(devy_r * sdy).reshape(1, 1)
    num2_blk = jnp.sum(devm_r * sdm).reshape(1, 1)
    sumw_blk = jnp.sum(sw_rows).reshape(1, 1)

    den1_o[...] = jnp.sum(devy_c * devy_c).reshape(1, 1)
    den2_o[...] = jnp.sum(devm_c * devm_c).reshape(1, 1)

    @pl.when(pid == 0)
    def _():
        num1_o[...] = num1_blk
        num2_o[...] = num2_blk
        sumw_o[...] = sumw_blk

    @pl.when(pid != 0)
    def _():
        num1_o[...] += num1_blk
        num2_o[...] += num2_blk
        sumw_o[...] += sumw_blk


@jax.jit
def kernel(target_x, target_y, mu):
    tx = jnp.squeeze(target_x, axis=0)
    ty = jnp.squeeze(jnp.squeeze(target_y, axis=0), axis=-1)
    m = jnp.squeeze(jnp.squeeze(mu, axis=0), axis=-1)
    x = tx[:, 0]
    y = tx[:, 1]
    sa = x * x + y * y
    bx = x.astype(jnp.bfloat16)
    by = y.astype(jnp.bfloat16)

    col = lambda v: v.reshape(1, N)
    row = lambda v: v.reshape(N, 1)

    grid = N // ROWS
    row_spec = pl.BlockSpec((ROWS, 1), lambda i: (i, 0))
    col_spec = pl.BlockSpec((1, N), lambda i: (0, 0))
    out_spec = pl.BlockSpec((1, 1), lambda i: (0, 0))
    scalar = jax.ShapeDtypeStruct((1, 1), jnp.float32)

    num1, num2, sumw, den1, den2 = pl.pallas_call(
        _moran_kernel,
        grid=(grid,),
        in_specs=[row_spec] * 5 + [col_spec] * 5,
        out_specs=[out_spec] * 5,
        out_shape=[scalar] * 5,
    )(row(bx), row(by), row(sa), row(ty), row(m),
      col(bx), col(by), col(sa), col(ty), col(m))

    nf = jnp.float32(N)
    moran_y = nf / sumw[0, 0] * num1[0, 0] / den1[0, 0]
    moran_mu = nf / sumw[0, 0] * num2[0, 0] / den2[0, 0]
    return (moran_y, moran_mu)
